# baseline (speedup 1.0000x reference)
import functools

import jax
import jax.numpy as jnp
import numpy as np
from jax import lax
from jax.experimental import pallas as pl
from jax.experimental.pallas import tpu as pltpu

_N_BINS = 15
_N = 100000
_C = 1000
_BLOCK_N = 400
_GRID = _N // _BLOCK_N
_NCORES = 2
_INNER = _GRID // _NCORES

_BOUNDS = [float(b) for b in
           np.linspace(0.0, 1.0, _N_BINS + 1).astype(np.float32)[:_N_BINS]]


def _ece_body(probs_ref, labels_ref, out_ref, acc_ref):
    i = pl.program_id(1)

    @pl.when(i == 0)
    def _init():
        acc_ref[...] = jnp.zeros_like(acc_ref)

    x = probs_ref[...]
    conf = jnp.max(x, axis=1, keepdims=True)
    col = lax.broadcasted_iota(jnp.int32, x.shape, 1)
    pred = jnp.min(jnp.where(x == conf, col, _C), axis=1, keepdims=True)
    lbl = labels_ref[0].reshape(_BLOCK_N, 1)
    correct = (pred == lbl).astype(jnp.float32)

    nbelow = jnp.zeros_like(conf, dtype=jnp.int32)
    for b in _BOUNDS:
        nbelow = nbelow + (conf > b).astype(jnp.int32)
    bin_idx = nbelow - 1
    lanes = lax.broadcasted_iota(jnp.int32, (_BLOCK_N, 128), 1)
    onehot = (bin_idx == lanes).astype(jnp.float32)

    acc_ref[0:1, :] += jnp.sum(onehot, axis=0, keepdims=True)
    acc_ref[1:2, :] += jnp.sum(onehot * conf, axis=0, keepdims=True)
    acc_ref[2:3, :] += jnp.sum(onehot * correct, axis=0, keepdims=True)

    @pl.when(i == _INNER - 1)
    def _fin():
        out_ref[0] = acc_ref[...]


@jax.jit
def _ece_pallas(probs, labels3):
    out = pl.pallas_call(
        _ece_body,
        grid=(_NCORES, _INNER),
        in_specs=[
            pl.BlockSpec((_BLOCK_N, _C), lambda o, i: (o * _INNER + i, 0)),
            pl.BlockSpec((1, 1, _BLOCK_N), lambda o, i: (o * _INNER + i, 0, 0)),
        ],
        out_specs=pl.BlockSpec((1, 8, 128), lambda o, i: (o, 0, 0)),
        out_shape=jax.ShapeDtypeStruct((_NCORES, 8, 128), jnp.float32),
        scratch_shapes=[pltpu.VMEM((8, 128), jnp.float32)],
        compiler_params=pltpu.CompilerParams(
            dimension_semantics=("parallel", "arbitrary"),
        ),
    )(probs, labels3)
    return out


def kernel(probs, labels, mode):
    del mode
    labels3 = labels.reshape(_GRID, 1, _BLOCK_N)
    out = _ece_pallas(probs, labels3)
    part = jnp.sum(out, axis=0)
    count = part[0, 0:_N_BINS]
    s_conf = part[1, 0:_N_BINS]
    s_corr = part[2, 0:_N_BINS]
    ece = jnp.sum(jnp.abs(s_conf - s_corr)).reshape(1)
    return (ece, s_corr, count)


# speedup vs baseline: 1.1815x; 1.1815x over previous
import jax
import jax.numpy as jnp
import numpy as np
from jax import lax
from jax.experimental import pallas as pl
from jax.experimental.pallas import tpu as pltpu

_N_BINS = 15
_N = 100000
_C = 1000
_NSTREAMS = 4
_BLOCK_N = 1000
_ROWS_PER_STEP = _NSTREAMS * _BLOCK_N
_GRID = _N // _ROWS_PER_STEP

_BOUNDS = [float(b) for b in
           np.linspace(0.0, 1.0, _N_BINS + 1).astype(np.float32)[:_N_BINS]]


def _partials(x, lbl):
    conf = jnp.max(x, axis=1, keepdims=True)
    col = lax.broadcasted_iota(jnp.int32, x.shape, 1)
    pred = jnp.min(jnp.where(x == conf, col, _C), axis=1, keepdims=True)
    correct = (pred == lbl).astype(jnp.float32)

    nbelow = jnp.zeros_like(conf, dtype=jnp.int32)
    for b in _BOUNDS:
        nbelow = nbelow + (conf > b).astype(jnp.int32)
    bin_idx = nbelow - 1
    lanes = lax.broadcasted_iota(jnp.int32, (x.shape[0], 128), 1)
    onehot = (bin_idx == lanes).astype(jnp.float32)

    return jnp.concatenate([
        jnp.sum(onehot, axis=0, keepdims=True),
        jnp.sum(onehot * conf, axis=0, keepdims=True),
        jnp.sum(onehot * correct, axis=0, keepdims=True),
    ], axis=0)


def _ece_body(p0, p1, p2, p3, l0, l1, l2, l3, out_ref, acc_ref):
    i = pl.program_id(0)

    @pl.when(i == 0)
    def _init():
        acc_ref[...] = jnp.zeros_like(acc_ref)

    for p_ref, l_ref in ((p0, l0), (p1, l1), (p2, l2), (p3, l3)):
        lbl = l_ref[0].reshape(_BLOCK_N, 1)
        acc_ref[0:3, :] += _partials(p_ref[...], lbl)

    @pl.when(i == _GRID - 1)
    def _fin():
        out_ref[...] = acc_ref[...]


@jax.jit
def _ece_pallas(probs, labels3):
    def pspec(k):
        return pl.BlockSpec((_BLOCK_N, _C),
                            lambda i, k=k: (_NSTREAMS * i + k, 0))

    def lspec(k):
        return pl.BlockSpec((1, 1, _BLOCK_N),
                            lambda i, k=k: (_NSTREAMS * i + k, 0, 0))

    out = pl.pallas_call(
        _ece_body,
        grid=(_GRID,),
        in_specs=[pspec(k) for k in range(_NSTREAMS)]
                 + [lspec(k) for k in range(_NSTREAMS)],
        out_specs=pl.BlockSpec((8, 128), lambda i: (0, 0)),
        out_shape=jax.ShapeDtypeStruct((8, 128), jnp.float32),
        scratch_shapes=[pltpu.VMEM((8, 128), jnp.float32)],
        compiler_params=pltpu.CompilerParams(
            dimension_semantics=("arbitrary",),
        ),
    )(*([probs] * _NSTREAMS), *([labels3] * _NSTREAMS))
    return out


def kernel(probs, labels, mode):
    del mode
    labels3 = labels.reshape(_N // _BLOCK_N, 1, _BLOCK_N)
    out = _ece_pallas(probs, labels3)
    count = out[0, 0:_N_BINS]
    s_conf = out[1, 0:_N_BINS]
    s_corr = out[2, 0:_N_BINS]
    ece = jnp.sum(jnp.abs(s_conf - s_corr)).reshape(1)
    return (ece, s_corr, count)


# speedup vs baseline: 2.8265x; 2.3922x over previous
import jax
import jax.numpy as jnp
import numpy as np
from jax import lax
from jax.experimental import pallas as pl
from jax.experimental.pallas import tpu as pltpu

_N_BINS = 15
_N = 100000
_C = 1000
_SUB = 8
_GRID = _C // _SUB

_BOUNDS = [float(b) for b in
           np.linspace(0.0, 1.0, _N_BINS + 1).astype(np.float32)[:_N_BINS]]


def _ece_body(pt_ref, labels_ref, out_ref, m_ref, i_ref):
    i = pl.program_id(0)
    x = pt_ref[...]

    @pl.when(i == 0)
    def _init():
        m_ref[...] = x
        i_ref[...] = jnp.zeros_like(i_ref)

    @pl.when(i > 0)
    def _upd():
        m = m_ref[...]
        p = x > m
        m_ref[...] = jnp.where(p, x, m)
        i_ref[...] = jnp.where(p, i, i_ref[...])

    @pl.when(i == _GRID - 1)
    def _fin():
        m = m_ref[...]
        conf = jnp.max(m, axis=0, keepdims=True)
        sub = lax.broadcasted_iota(jnp.int32, m.shape, 0)
        cls = i_ref[...] * _SUB + sub
        pred = jnp.min(jnp.where(m == conf, cls, _C), axis=0, keepdims=True)
        correct = (pred == labels_ref[...]).astype(jnp.float32)

        nbelow = jnp.zeros_like(conf, dtype=jnp.int32)
        for b in _BOUNDS:
            nbelow = nbelow + (conf > b).astype(jnp.int32)
        bin_idx = nbelow - 1

        zero = jnp.zeros_like(conf)
        for j in range(_N_BINS):
            sel = bin_idx == j
            cnt = jnp.sum(jnp.where(sel, 1.0, 0.0), axis=1, keepdims=True)
            s_cf = jnp.sum(jnp.where(sel, conf, zero), axis=1, keepdims=True)
            s_co = jnp.sum(jnp.where(sel, correct, zero), axis=1, keepdims=True)
            out_ref[0:1, j:j + 1] = cnt
            out_ref[1:2, j:j + 1] = s_cf
            out_ref[2:3, j:j + 1] = s_co


@jax.jit
def _ece_pallas(pt, labels2):
    out = pl.pallas_call(
        _ece_body,
        grid=(_GRID,),
        in_specs=[
            pl.BlockSpec((_SUB, _N), lambda i: (i, 0)),
            pl.BlockSpec((1, _N), lambda i: (0, 0)),
        ],
        out_specs=pl.BlockSpec((8, 128), lambda i: (0, 0)),
        out_shape=jax.ShapeDtypeStruct((8, 128), jnp.float32),
        scratch_shapes=[pltpu.VMEM((_SUB, _N), jnp.float32),
                        pltpu.VMEM((_SUB, _N), jnp.int32)],
        compiler_params=pltpu.CompilerParams(
            dimension_semantics=("arbitrary",),
        ),
    )(pt, labels2)
    return out


def kernel(probs, labels, mode):
    del mode
    pt = probs.T
    labels2 = labels.reshape(1, _N)
    out = _ece_pallas(pt, labels2)
    count = out[0, 0:_N_BINS]
    s_conf = out[1, 0:_N_BINS]
    s_corr = out[2, 0:_N_BINS]
    ece = jnp.sum(jnp.abs(s_conf - s_corr)).reshape(1)
    return (ece, s_corr, count)


# speedup vs baseline: 3.0198x; 1.0684x over previous
import jax
import jax.numpy as jnp
import numpy as np
from jax import lax
from jax.experimental import pallas as pl
from jax.experimental.pallas import tpu as pltpu

_N_BINS = 15
_N = 100000
_C = 1000
_SUB = 8
_NSTREAMS = 5
_TROWS = _C // _SUB
_GRID = _TROWS // _NSTREAMS

_BOUNDS = [float(b) for b in
           np.linspace(0.0, 1.0, _N_BINS + 1).astype(np.float32)[:_N_BINS]]


def _ece_body(p0, p1, p2, p3, p4, labels_ref, out_ref, m_ref, i_ref):
    i = pl.program_id(0)
    refs = (p0, p1, p2, p3, p4)

    def upd(x, tile_row):
        m = m_ref[...]
        p = x > m
        m_ref[...] = jnp.where(p, x, m)
        i_ref[...] = jnp.where(p, tile_row, i_ref[...])

    @pl.when(i == 0)
    def _init():
        m_ref[...] = p0[...]
        i_ref[...] = jnp.zeros_like(i_ref)
        for k in range(1, _NSTREAMS):
            upd(refs[k][...], k)

    @pl.when(i > 0)
    def _upd():
        for k in range(_NSTREAMS):
            upd(refs[k][...], _NSTREAMS * i + k)

    @pl.when(i == _GRID - 1)
    def _fin():
        m = m_ref[...]
        conf = jnp.max(m, axis=0, keepdims=True)
        sub = lax.broadcasted_iota(jnp.int32, m.shape, 0)
        cls = i_ref[...] * _SUB + sub
        pred = jnp.min(jnp.where(m == conf, cls, _C), axis=0, keepdims=True)
        correct = (pred == labels_ref[...]).astype(jnp.float32)

        nbelow = jnp.zeros_like(conf, dtype=jnp.int32)
        for b in _BOUNDS:
            nbelow = nbelow + (conf > b).astype(jnp.int32)
        bin_idx = nbelow - 1

        zero = jnp.zeros_like(conf)
        for j in range(_N_BINS):
            sel = bin_idx == j
            cnt = jnp.sum(jnp.where(sel, 1.0, 0.0), axis=1, keepdims=True)
            s_cf = jnp.sum(jnp.where(sel, conf, zero), axis=1, keepdims=True)
            s_co = jnp.sum(jnp.where(sel, correct, zero), axis=1, keepdims=True)
            out_ref[0:1, j:j + 1] = cnt
            out_ref[1:2, j:j + 1] = s_cf
            out_ref[2:3, j:j + 1] = s_co


@jax.jit
def _ece_pallas(pt, labels2):
    def pspec(k):
        return pl.BlockSpec((_SUB, _N), lambda i, k=k: (_NSTREAMS * i + k, 0))

    out = pl.pallas_call(
        _ece_body,
        grid=(_GRID,),
        in_specs=[pspec(k) for k in range(_NSTREAMS)]
                 + [pl.BlockSpec((1, _N), lambda i: (0, 0))],
        out_specs=pl.BlockSpec((8, 128), lambda i: (0, 0)),
        out_shape=jax.ShapeDtypeStruct((8, 128), jnp.float32),
        scratch_shapes=[pltpu.VMEM((_SUB, _N), jnp.float32),
                        pltpu.VMEM((_SUB, _N), jnp.int32)],
        compiler_params=pltpu.CompilerParams(
            dimension_semantics=("arbitrary",),
        ),
    )(*([pt] * _NSTREAMS), labels2)
    return out


def kernel(probs, labels, mode):
    del mode
    pt = probs.T
    labels2 = labels.reshape(1, _N)
    out = _ece_pallas(pt, labels2)
    count = out[0, 0:_N_BINS]
    s_conf = out[1, 0:_N_BINS]
    s_corr = out[2, 0:_N_BINS]
    ece = jnp.sum(jnp.abs(s_conf - s_corr)).reshape(1)
    return (ece, s_corr, count)
